# baseline (speedup 1.0000x reference)
import functools

import jax
import jax.numpy as jnp
from jax import lax
from jax.experimental import pallas as pl
from jax.experimental.pallas import tpu as pltpu
from jax.experimental.pallas import tpu_sc as plsc

VOCAB = 100000
EMB = 128
HID = 1024
SCALE = float(EMB) ** 0.5


_NW = 32
_CHUNK = 400


def _sc_gather_body(table_hbm, idx_hbm, out_hbm, idx_a, idx_b, rows_a,
                    rows_b, sem_a, sem_b, *, n_tokens):
    b_per_w = n_tokens // _NW
    n_chunks = b_per_w // _CHUNK
    wid = lax.axis_index("s") * 2 + lax.axis_index("c")
    base = wid * b_per_w

    idx_bufs = [idx_a, idx_b]
    row_bufs = [rows_a, rows_b]
    sems = [sem_a, sem_b]
    copies = [None, None]

    pltpu.sync_copy(idx_hbm.at[pl.ds(base, _CHUNK)], idx_a)
    copies[0] = pltpu.async_copy(table_hbm.at[idx_a], rows_a, sem_a)
    for i in range(n_chunks):
        cur, nxt = i & 1, (i + 1) & 1
        if i + 1 < n_chunks:
            start = base + (i + 1) * _CHUNK
            pltpu.sync_copy(idx_hbm.at[pl.ds(start, _CHUNK)], idx_bufs[nxt])
            copies[nxt] = pltpu.async_copy(
                table_hbm.at[idx_bufs[nxt]], row_bufs[nxt], sems[nxt])
        copies[cur].wait()
        pltpu.sync_copy(row_bufs[cur], out_hbm.at[pl.ds(base + i * _CHUNK, _CHUNK)])


def _sc_gather(table, idx_flat):
    n_tokens = idx_flat.shape[0]
    mesh = plsc.VectorSubcoreMesh(core_axis_name="c", subcore_axis_name="s")
    return pl.kernel(
        functools.partial(_sc_gather_body, n_tokens=n_tokens),
        out_type=jax.ShapeDtypeStruct((n_tokens, EMB), jnp.float32),
        mesh=mesh,
        scratch_types=[
            pltpu.VMEM((_CHUNK,), jnp.int32),
            pltpu.VMEM((_CHUNK,), jnp.int32),
            pltpu.VMEM((_CHUNK, EMB), jnp.float32),
            pltpu.VMEM((_CHUNK, EMB), jnp.float32),
            pltpu.SemaphoreType.DMA,
            pltpu.SemaphoreType.DMA,
        ],
    )(table, idx_flat)


_TOK_BLK = 4096
_NSLICE = 2


def _tc_project_body(emb_ref, ids_ref, w_ref, out_ref):
    mask = ids_ref[...] != 0
    e = jnp.where(mask, emb_ref[...], 0.0) * SCALE
    out_ref[...] = jnp.dot(e, w_ref[...], preferred_element_type=jnp.float32)


def _tc_body_chained(prev_ref, emb_ref, ids_ref, w_ref, out_ref):
    del prev_ref
    _tc_project_body(emb_ref, ids_ref, w_ref, out_ref)


def _tc_project_slice(prev, gathered_j, ids_col_j, w, blk_base, n_tokens):
    nblk = gathered_j.shape[0] // _TOK_BLK
    slice_specs = [
        pl.BlockSpec((_TOK_BLK, EMB), lambda i: (i, 0)),
        pl.BlockSpec((_TOK_BLK, 1), lambda i: (i, 0)),
        pl.BlockSpec((EMB, HID), lambda i: (0, 0)),
    ]
    out_spec = pl.BlockSpec((_TOK_BLK, HID), lambda i: (blk_base + i, 0))
    out_shape = jax.ShapeDtypeStruct((n_tokens, HID), jnp.float32)
    if prev is None:
        return pl.pallas_call(
            _tc_project_body,
            grid=(nblk,),
            in_specs=slice_specs,
            out_specs=out_spec,
            out_shape=out_shape,
        )(gathered_j, ids_col_j, w)
    return pl.pallas_call(
        _tc_body_chained,
        grid=(nblk,),
        in_specs=[pl.BlockSpec(memory_space=pl.ANY)] + slice_specs,
        out_specs=out_spec,
        out_shape=out_shape,
        input_output_aliases={0: 0},
    )(prev, gathered_j, ids_col_j, w)


def kernel(inputs, shared_weights, map_weights):
    b, l = inputs.shape
    n_tokens = b * l
    idx_flat = inputs.reshape(-1)
    sl = n_tokens // _NSLICE
    gathered = [
        _sc_gather(shared_weights, lax.slice(idx_flat, (j * sl,), ((j + 1) * sl,)))
        for j in range(_NSLICE)
    ]
    ids_col = idx_flat.reshape(-1, 1)
    out = None
    for j in range(_NSLICE):
        out = _tc_project_slice(
            out, gathered[j],
            lax.slice(ids_col, (j * sl, 0), ((j + 1) * sl, 1)),
            map_weights,
            blk_base=j * (sl // _TOK_BLK),
            n_tokens=n_tokens,
        )
    return out.reshape(b, l, HID)


# speedup vs baseline: 1.1100x; 1.1100x over previous
import functools

import jax
import jax.numpy as jnp
from jax import lax
from jax.experimental import pallas as pl
from jax.experimental.pallas import tpu as pltpu
from jax.experimental.pallas import tpu_sc as plsc

VOCAB = 100000
EMB = 128
HID = 1024
SCALE = float(EMB) ** 0.5


_NW = 32
_CHUNK = 400


def _sc_gather_body(table_hbm, idx_hbm, out_hbm, idx_a, idx_b, rows_a,
                    rows_b, sem_a, sem_b, *, n_tokens):
    b_per_w = n_tokens // _NW
    n_chunks = b_per_w // _CHUNK
    wid = lax.axis_index("s") * 2 + lax.axis_index("c")
    base = wid * b_per_w

    idx_bufs = [idx_a, idx_b]
    row_bufs = [rows_a, rows_b]
    sems = [sem_a, sem_b]
    copies = [None, None]

    pltpu.sync_copy(idx_hbm.at[pl.ds(base, _CHUNK)], idx_a)
    copies[0] = pltpu.async_copy(table_hbm.at[idx_a], rows_a, sem_a)
    for i in range(n_chunks):
        cur, nxt = i & 1, (i + 1) & 1
        if i + 1 < n_chunks:
            start = base + (i + 1) * _CHUNK
            pltpu.sync_copy(idx_hbm.at[pl.ds(start, _CHUNK)], idx_bufs[nxt])
            copies[nxt] = pltpu.async_copy(
                table_hbm.at[idx_bufs[nxt]], row_bufs[nxt], sems[nxt])
        copies[cur].wait()
        pltpu.sync_copy(row_bufs[cur], out_hbm.at[pl.ds(base + i * _CHUNK, _CHUNK)])


def _sc_gather(table, idx_flat):
    n_tokens = idx_flat.shape[0]
    mesh = plsc.VectorSubcoreMesh(core_axis_name="c", subcore_axis_name="s")
    return pl.kernel(
        functools.partial(_sc_gather_body, n_tokens=n_tokens),
        out_type=jax.ShapeDtypeStruct((n_tokens, EMB), jnp.float32),
        mesh=mesh,
        scratch_types=[
            pltpu.VMEM((_CHUNK,), jnp.int32),
            pltpu.VMEM((_CHUNK,), jnp.int32),
            pltpu.VMEM((_CHUNK, EMB), jnp.float32),
            pltpu.VMEM((_CHUNK, EMB), jnp.float32),
            pltpu.SemaphoreType.DMA,
            pltpu.SemaphoreType.DMA,
        ],
    )(table, idx_flat)


_TOK_BLK = 4096
_NSLICE = 1


def _tc_project_body(emb_ref, ids_ref, w_ref, out_ref):
    mask = ids_ref[...] != 0
    e = jnp.where(mask, emb_ref[...], 0.0) * SCALE
    out_ref[...] = jnp.dot(e.astype(jnp.bfloat16),
                           w_ref[...].astype(jnp.bfloat16),
                           preferred_element_type=jnp.float32)


def _tc_body_chained(prev_ref, emb_ref, ids_ref, w_ref, out_ref):
    del prev_ref
    _tc_project_body(emb_ref, ids_ref, w_ref, out_ref)


def _tc_project_slice(prev, gathered_j, ids_col_j, w, blk_base, n_tokens):
    nblk = gathered_j.shape[0] // _TOK_BLK
    slice_specs = [
        pl.BlockSpec((_TOK_BLK, EMB), lambda i: (i, 0)),
        pl.BlockSpec((_TOK_BLK, 1), lambda i: (i, 0)),
        pl.BlockSpec((EMB, HID), lambda i: (0, 0)),
    ]
    out_spec = pl.BlockSpec((_TOK_BLK, HID), lambda i: (blk_base + i, 0))
    out_shape = jax.ShapeDtypeStruct((n_tokens, HID), jnp.float32)
    if prev is None:
        return pl.pallas_call(
            _tc_project_body,
            grid=(nblk,),
            in_specs=slice_specs,
            out_specs=out_spec,
            out_shape=out_shape,
        )(gathered_j, ids_col_j, w)
    return pl.pallas_call(
        _tc_body_chained,
        grid=(nblk,),
        in_specs=[pl.BlockSpec(memory_space=pl.ANY)] + slice_specs,
        out_specs=out_spec,
        out_shape=out_shape,
        input_output_aliases={0: 0},
    )(prev, gathered_j, ids_col_j, w)


def kernel(inputs, shared_weights, map_weights):
    b, l = inputs.shape
    n_tokens = b * l
    idx_flat = inputs.reshape(-1)
    sl = n_tokens // _NSLICE
    gathered = [
        _sc_gather(shared_weights, lax.slice(idx_flat, (j * sl,), ((j + 1) * sl,)))
        for j in range(_NSLICE)
    ]
    ids_col = idx_flat.reshape(-1, 1)
    out = None
    for j in range(_NSLICE):
        out = _tc_project_slice(
            out, gathered[j],
            lax.slice(ids_col, (j * sl, 0), ((j + 1) * sl, 1)),
            map_weights,
            blk_base=j * (sl // _TOK_BLK),
            n_tokens=n_tokens,
        )
    return out.reshape(b, l, HID)


# speedup vs baseline: 1.1131x; 1.0028x over previous
import functools

import jax
import jax.numpy as jnp
from jax import lax
from jax.experimental import pallas as pl
from jax.experimental.pallas import tpu as pltpu
from jax.experimental.pallas import tpu_sc as plsc

VOCAB = 100000
EMB = 128
HID = 1024
SCALE = float(EMB) ** 0.5


_NW = 32
_CHUNK = 256
_NBUF = 3


def _sc_gather_body(table_hbm, idx_hbm, out_hbm, idx_all,
                    rows_0, rows_1, rows_2,
                    gs_0, gs_1, gs_2, ws_0, ws_1, ws_2, *, n_tokens):
    b_per_w = n_tokens // _NW
    n_chunks = b_per_w // _CHUNK
    wid = lax.axis_index("s") * 2 + lax.axis_index("c")
    base = wid * b_per_w

    row_bufs = [rows_0, rows_1, rows_2]
    gsems = [gs_0, gs_1, gs_2]
    wsems = [ws_0, ws_1, ws_2]
    gcp = [None] * _NBUF
    wcp = [None] * _NBUF

    pltpu.sync_copy(idx_hbm.at[pl.ds(base, b_per_w)], idx_all)

    for i in range(n_chunks):
        k = i % _NBUF
        if wcp[k] is not None:
            wcp[k].wait()
        gcp[k] = pltpu.async_copy(
            table_hbm.at[idx_all.at[pl.ds(i * _CHUNK, _CHUNK)]],
            row_bufs[k], gsems[k])
        if i >= 1:
            kp = (i - 1) % _NBUF
            gcp[kp].wait()
            wcp[kp] = pltpu.async_copy(
                row_bufs[kp],
                out_hbm.at[pl.ds(base + (i - 1) * _CHUNK, _CHUNK)],
                wsems[kp])
    kl = (n_chunks - 1) % _NBUF
    gcp[kl].wait()
    wcp[kl] = pltpu.async_copy(
        row_bufs[kl],
        out_hbm.at[pl.ds(base + (n_chunks - 1) * _CHUNK, _CHUNK)],
        wsems[kl])
    for k in range(_NBUF):
        if wcp[k] is not None:
            wcp[k].wait()


def _sc_gather(table, idx_flat):
    n_tokens = idx_flat.shape[0]
    width = table.shape[1]
    mesh = plsc.VectorSubcoreMesh(core_axis_name="c", subcore_axis_name="s")
    return pl.kernel(
        functools.partial(_sc_gather_body, n_tokens=n_tokens),
        out_type=jax.ShapeDtypeStruct((n_tokens, width), table.dtype),
        mesh=mesh,
        scratch_types=[
            pltpu.VMEM((n_tokens // _NW,), jnp.int32),
            pltpu.VMEM((_CHUNK, width), table.dtype),
            pltpu.VMEM((_CHUNK, width), table.dtype),
            pltpu.VMEM((_CHUNK, width), table.dtype),
            pltpu.SemaphoreType.DMA,
            pltpu.SemaphoreType.DMA,
            pltpu.SemaphoreType.DMA,
            pltpu.SemaphoreType.DMA,
            pltpu.SemaphoreType.DMA,
            pltpu.SemaphoreType.DMA,
        ],
    )(table, idx_flat)


_TOK_BLK = 4096
_NSLICE = 1


def _tc_project_body(emb_ref, ids_ref, w_ref, out_ref):
    mask = (ids_ref[...] != 0).astype(jnp.float32)
    e = emb_ref[...] * (mask * SCALE)
    out_ref[...] = jnp.dot(e, w_ref[...], preferred_element_type=jnp.float32)


def _tc_body_chained(prev_ref, emb_ref, ids_ref, w_ref, out_ref):
    del prev_ref
    _tc_project_body(emb_ref, ids_ref, w_ref, out_ref)


def _tc_project_slice(prev, gathered_j, ids_col_j, w, blk_base, n_tokens):
    nblk = gathered_j.shape[0] // _TOK_BLK
    slice_specs = [
        pl.BlockSpec((_TOK_BLK, EMB), lambda i: (i, 0)),
        pl.BlockSpec((_TOK_BLK, 1), lambda i: (i, 0)),
        pl.BlockSpec((EMB, HID), lambda i: (0, 0)),
    ]
    out_spec = pl.BlockSpec((_TOK_BLK, HID), lambda i: (blk_base + i, 0))
    out_shape = jax.ShapeDtypeStruct((n_tokens, HID), jnp.float32)
    if prev is None:
        return pl.pallas_call(
            _tc_project_body,
            grid=(nblk,),
            in_specs=slice_specs,
            out_specs=out_spec,
            out_shape=out_shape,
        )(gathered_j, ids_col_j, w)
    return pl.pallas_call(
        _tc_body_chained,
        grid=(nblk,),
        in_specs=[pl.BlockSpec(memory_space=pl.ANY)] + slice_specs,
        out_specs=out_spec,
        out_shape=out_shape,
        input_output_aliases={0: 0},
    )(prev, gathered_j, ids_col_j, w)


def kernel(inputs, shared_weights, map_weights):
    b, l = inputs.shape
    n_tokens = b * l
    idx_flat = inputs.reshape(-1)
    sl = n_tokens // _NSLICE
    gathered = [
        _sc_gather(shared_weights, lax.slice(idx_flat, (j * sl,), ((j + 1) * sl,)))
        for j in range(_NSLICE)
    ]
    ids_col = idx_flat.reshape(-1, 1)
    out = None
    for j in range(_NSLICE):
        out = _tc_project_slice(
            out, gathered[j],
            lax.slice(ids_col, (j * sl, 0), ((j + 1) * sl, 1)),
            map_weights,
            blk_base=j * (sl // _TOK_BLK),
            n_tokens=n_tokens,
        )
    return out.reshape(b, l, HID)


# speedup vs baseline: 1.3098x; 1.1767x over previous
import functools

import jax
import jax.numpy as jnp
from jax import lax
from jax.experimental import pallas as pl
from jax.experimental.pallas import tpu as pltpu
from jax.experimental.pallas import tpu_sc as plsc

VOCAB = 100000
EMB = 128
HID = 1024
SCALE = float(EMB) ** 0.5


_NW = 32
_CHUNK = 256
_NBUF = 3


def _sc_gather_body(table_hbm, idx_hbm, out_hbm, idx_all,
                    rows_0, rows_1, rows_2,
                    gs_0, gs_1, gs_2, ws_0, ws_1, ws_2, *, n_tokens):
    b_per_w = n_tokens // _NW
    n_chunks = b_per_w // _CHUNK
    wid = lax.axis_index("s") * 2 + lax.axis_index("c")
    base = wid * b_per_w

    row_bufs = [rows_0, rows_1, rows_2]
    gsems = [gs_0, gs_1, gs_2]
    wsems = [ws_0, ws_1, ws_2]
    gcp = [None] * _NBUF
    wcp = [None] * _NBUF

    pltpu.sync_copy(idx_hbm.at[pl.ds(base, b_per_w)], idx_all)

    for i in range(n_chunks):
        k = i % _NBUF
        if wcp[k] is not None:
            wcp[k].wait()
        gcp[k] = pltpu.async_copy(
            table_hbm.at[idx_all.at[pl.ds(i * _CHUNK, _CHUNK)]],
            row_bufs[k], gsems[k])
        if i >= 1:
            kp = (i - 1) % _NBUF
            gcp[kp].wait()
            wcp[kp] = pltpu.async_copy(
                row_bufs[kp],
                out_hbm.at[pl.ds(base + (i - 1) * _CHUNK, _CHUNK)],
                wsems[kp])
    kl = (n_chunks - 1) % _NBUF
    gcp[kl].wait()
    wcp[kl] = pltpu.async_copy(
        row_bufs[kl],
        out_hbm.at[pl.ds(base + (n_chunks - 1) * _CHUNK, _CHUNK)],
        wsems[kl])
    for k in range(_NBUF):
        if wcp[k] is not None:
            wcp[k].wait()


def _sc_gather(table, idx_flat):
    n_tokens = idx_flat.shape[0]
    width = table.shape[1]
    mesh = plsc.VectorSubcoreMesh(core_axis_name="c", subcore_axis_name="s")
    return pl.kernel(
        functools.partial(_sc_gather_body, n_tokens=n_tokens),
        out_type=jax.ShapeDtypeStruct((n_tokens, width), table.dtype),
        mesh=mesh,
        scratch_types=[
            pltpu.VMEM((n_tokens // _NW,), jnp.int32),
            pltpu.VMEM((_CHUNK, width), table.dtype),
            pltpu.VMEM((_CHUNK, width), table.dtype),
            pltpu.VMEM((_CHUNK, width), table.dtype),
            pltpu.SemaphoreType.DMA,
            pltpu.SemaphoreType.DMA,
            pltpu.SemaphoreType.DMA,
            pltpu.SemaphoreType.DMA,
            pltpu.SemaphoreType.DMA,
            pltpu.SemaphoreType.DMA,
        ],
    )(table, idx_flat)


_TOK_BLK = 4096
_NSLICE = 1


def _tc_project_body(emb_ref, ids_ref, w_ref, out_ref):
    rows = _TOK_BLK // 128
    mask = (ids_ref[...] != 0).astype(jnp.float32) * SCALE
    e = emb_ref[...].reshape(rows, 128, EMB) * mask[:, :, None]
    out_ref[...] = jnp.dot(e.reshape(_TOK_BLK, EMB), w_ref[...],
                           preferred_element_type=jnp.float32)


def _tc_body_chained(prev_ref, emb_ref, ids_ref, w_ref, out_ref):
    del prev_ref
    _tc_project_body(emb_ref, ids_ref, w_ref, out_ref)


def _tc_project_slice(prev, gathered_j, ids_col_j, w, blk_base, n_tokens):
    nblk = gathered_j.shape[0] // _TOK_BLK
    slice_specs = [
        pl.BlockSpec((_TOK_BLK, EMB), lambda i: (i, 0)),
        pl.BlockSpec((_TOK_BLK // 128, 128), lambda i: (i, 0)),
        pl.BlockSpec((EMB, HID), lambda i: (0, 0)),
    ]
    out_spec = pl.BlockSpec((_TOK_BLK, HID), lambda i: (blk_base + i, 0))
    out_shape = jax.ShapeDtypeStruct((n_tokens, HID), jnp.float32)
    if prev is None:
        return pl.pallas_call(
            _tc_project_body,
            grid=(nblk,),
            in_specs=slice_specs,
            out_specs=out_spec,
            out_shape=out_shape,
        )(gathered_j, ids_col_j, w)
    return pl.pallas_call(
        _tc_body_chained,
        grid=(nblk,),
        in_specs=[pl.BlockSpec(memory_space=pl.ANY)] + slice_specs,
        out_specs=out_spec,
        out_shape=out_shape,
        input_output_aliases={0: 0},
    )(prev, gathered_j, ids_col_j, w)


def kernel(inputs, shared_weights, map_weights):
    b, l = inputs.shape
    n_tokens = b * l
    idx_flat = inputs.reshape(-1)
    sl = n_tokens // _NSLICE
    gathered = [
        _sc_gather(shared_weights, lax.slice(idx_flat, (j * sl,), ((j + 1) * sl,)))
        for j in range(_NSLICE)
    ]
    ids_pack = idx_flat.reshape(-1, 128)
    out = None
    for j in range(_NSLICE):
        out = _tc_project_slice(
            out, gathered[j],
            lax.slice(ids_pack, (j * sl // 128, 0), ((j + 1) * sl // 128, 128)),
            map_weights,
            blk_base=j * (sl // _TOK_BLK),
            n_tokens=n_tokens,
        )
    return out.reshape(b, l, HID)


# speedup vs baseline: 1.3124x; 1.0020x over previous
import functools

import jax
import jax.numpy as jnp
from jax import lax
from jax.experimental import pallas as pl
from jax.experimental.pallas import tpu as pltpu
from jax.experimental.pallas import tpu_sc as plsc

VOCAB = 100000
EMB = 128
HID = 1024
SCALE = float(EMB) ** 0.5


_NW = 32
_CHUNK = 256
_NBUF = 3


def _sc_gather_body(table_hbm, idx_hbm, out_hbm, idx_all,
                    rows_0, rows_1, rows_2,
                    gs_0, gs_1, gs_2, ws_0, ws_1, ws_2, *, n_tokens):
    b_per_w = n_tokens // _NW
    n_chunks = b_per_w // _CHUNK
    wid = lax.axis_index("s") * 2 + lax.axis_index("c")
    base = wid * b_per_w

    row_bufs = [rows_0, rows_1, rows_2]
    gsems = [gs_0, gs_1, gs_2]
    wsems = [ws_0, ws_1, ws_2]
    gcp = [None] * _NBUF
    wcp = [None] * _NBUF

    pltpu.sync_copy(idx_hbm.at[pl.ds(base, b_per_w)], idx_all)

    for i in range(n_chunks):
        k = i % _NBUF
        if wcp[k] is not None:
            wcp[k].wait()
        gcp[k] = pltpu.async_copy(
            table_hbm.at[idx_all.at[pl.ds(i * _CHUNK, _CHUNK)]],
            row_bufs[k], gsems[k])
        if i >= 1:
            kp = (i - 1) % _NBUF
            gcp[kp].wait()
            wcp[kp] = pltpu.async_copy(
                row_bufs[kp],
                out_hbm.at[pl.ds(base + (i - 1) * _CHUNK, _CHUNK)],
                wsems[kp])
    kl = (n_chunks - 1) % _NBUF
    gcp[kl].wait()
    wcp[kl] = pltpu.async_copy(
        row_bufs[kl],
        out_hbm.at[pl.ds(base + (n_chunks - 1) * _CHUNK, _CHUNK)],
        wsems[kl])
    for k in range(_NBUF):
        if wcp[k] is not None:
            wcp[k].wait()


def _sc_gather(table, idx_flat):
    n_tokens = idx_flat.shape[0]
    width = table.shape[1]
    mesh = plsc.VectorSubcoreMesh(core_axis_name="c", subcore_axis_name="s")
    return pl.kernel(
        functools.partial(_sc_gather_body, n_tokens=n_tokens),
        out_type=jax.ShapeDtypeStruct((n_tokens, width), table.dtype),
        mesh=mesh,
        scratch_types=[
            pltpu.VMEM((n_tokens // _NW,), jnp.int32),
            pltpu.VMEM((_CHUNK, width), table.dtype),
            pltpu.VMEM((_CHUNK, width), table.dtype),
            pltpu.VMEM((_CHUNK, width), table.dtype),
            pltpu.SemaphoreType.DMA,
            pltpu.SemaphoreType.DMA,
            pltpu.SemaphoreType.DMA,
            pltpu.SemaphoreType.DMA,
            pltpu.SemaphoreType.DMA,
            pltpu.SemaphoreType.DMA,
        ],
    )(table, idx_flat)


_TOK_BLK = 4096


def _tc_project_body(emb_ref, ids_ref, w_ref, out_ref):
    rows = _TOK_BLK // 128
    mask = (ids_ref[...] != 0).astype(jnp.float32) * SCALE
    e = emb_ref[...].reshape(rows, 128, EMB) * mask[:, :, None]
    out_ref[...] = jnp.dot(e.reshape(_TOK_BLK, EMB), w_ref[...],
                           preferred_element_type=jnp.float32)


def _tc_project(gathered, ids_pack, w):
    n_tokens = gathered.shape[0]
    return pl.pallas_call(
        _tc_project_body,
        grid=(n_tokens // _TOK_BLK,),
        in_specs=[
            pl.BlockSpec((_TOK_BLK, EMB), lambda i: (i, 0)),
            pl.BlockSpec((_TOK_BLK // 128, 128), lambda i: (i, 0)),
            pl.BlockSpec((EMB, HID), lambda i: (0, 0)),
        ],
        out_specs=pl.BlockSpec((_TOK_BLK, HID), lambda i: (i, 0)),
        out_shape=jax.ShapeDtypeStruct((n_tokens, HID), jnp.float32),
    )(gathered, ids_pack, w)


def kernel(inputs, shared_weights, map_weights):
    b, l = inputs.shape
    idx_flat = inputs.reshape(-1)
    gathered = _sc_gather(shared_weights, idx_flat)
    out2d = _tc_project(gathered, idx_flat.reshape(-1, 128), map_weights)
    return out2d.reshape(b, l, HID)
